# baseline (speedup 1.0000x reference)
import functools

import jax
import jax.numpy as jnp
from jax import lax
from jax.experimental import pallas as pl
from jax.experimental.pallas import tpu as pltpu
from jax.experimental.pallas import tpu_sc as plsc

L = 16


def _rsqrt_newton(ss):
    i = lax.bitcast_convert_type(ss, jnp.int32)
    i = jnp.int32(0x5F3759DF) - lax.shift_right_logical(i, 1)
    y = lax.bitcast_convert_type(i, jnp.float32)
    half = ss * 0.5
    for _ in range(3):
        y = y * (1.5 - (half * y) * y)
    return y


def _make_sc_kernel(B, N, G):
    info = plsc.get_sparse_core_info()
    NC, NS = info.num_cores, info.num_subcores
    NW = NC * NS
    total = B * N
    K = total // NW
    tiles_per_b = N // K
    CS = 256
    NCH = K // CS
    VC = CS // L
    G3 = G * G * G
    mesh = plsc.VectorSubcoreMesh(core_axis_name="c", subcore_axis_name="s")

    COFF = [(i * G * G + j * G + k) for i in (0, 1) for j in (0, 1) for k in (0, 1)]

    @functools.partial(
        pl.kernel,
        mesh=mesh,
        out_type=[
            jax.ShapeDtypeStruct((B * N,), jnp.float32),
            jax.ShapeDtypeStruct((B * 3 * N,), jnp.float32),
            jax.ShapeDtypeStruct((NW * L,), jnp.float32),
        ],
        scratch_types=(
            [pltpu.VMEM((K,), jnp.float32) for _ in range(3)]
            + [pltpu.VMEM((K,), jnp.float32) for _ in range(3)]
            + [pltpu.VMEM((K,), jnp.int32) for _ in range(8)]
            + [pltpu.VMEM((K,), jnp.float32) for _ in range(8)]
            + [pltpu.VMEM((K,), jnp.float32)]
            + [pltpu.VMEM((K,), jnp.float32) for _ in range(3)]
            + [pltpu.VMEM((9 * L,), jnp.float32)]
            + [pltpu.VMEM((L,), jnp.float32)]
            + [pltpu.SemaphoreType.DMA for _ in range(8)]
        ),
    )
    def sc_kernel(pss_hbm, grid_hbm, params_hbm,
                  dss_hbm, nss_hbm, lpart_hbm,
                  cx, cy, cz, fx_, fy_, fz_,
                  i0, i1, i2, i3, i4, i5, i6, i7,
                  g0, g1, g2, g3, g4, g5, g6, g7,
                  dbuf, n0b, n1b, n2b, pbuf, lbuf,
                  sem0, sem1, sem2, sem3, sem4, sem5, sem6, sem7):
        sems = (sem0, sem1, sem2, sem3, sem4, sem5, sem6, sem7)
        cbuf = (cx, cy, cz)
        fbuf = (fx_, fy_, fz_)
        idxb = (i0, i1, i2, i3, i4, i5, i6, i7)
        corn = (g0, g1, g2, g3, g4, g5, g6, g7)

        wid = lax.axis_index("s") * NC + lax.axis_index("c")
        b = wid // tiles_per_b
        n0 = (wid % tiles_per_b) * K

        for a in range(3):
            pltpu.sync_copy(pss_hbm.at[pl.ds((b * 3 + a) * N + n0, K)], cbuf[a])
        pltpu.sync_copy(params_hbm.at[pl.ds(b * 9 * L, 9 * L)], pbuf)

        gbase = b * G3

        def phase_a(v, _):
            sl = pl.ds(v * L, L)
            ib = []
            for a in range(3):
                p = cbuf[a][sl]
                f0 = jnp.maximum(
                    jnp.minimum((p - pbuf[pl.ds(a * L, L)]) * pbuf[pl.ds((3 + a) * L, L)],
                                pbuf[pl.ds((6 + a) * L, L)]), 0.0)
                ia = f0.astype(jnp.int32)
                fbuf[a][sl] = f0 - ia.astype(jnp.float32)
                ib.append(ia)
            flat = (ib[0] * G + ib[1]) * G + ib[2] + gbase
            for c in range(8):
                idxb[c][sl] = flat + COFF[c]
            return 0

        def fire(ch):
            cs = pl.ds(ch * CS, CS)
            for c in range(8):
                pltpu.async_copy(grid_hbm.at[idxb[c].at[cs]], corn[c].at[cs],
                                 sems[ch])

        def drain(ch):
            cs = pl.ds(ch * CS, CS)
            for c in range(8):
                pltpu.make_async_copy(
                    grid_hbm.at[idxb[c].at[cs]], corn[c].at[cs],
                    sems[ch]).wait()

        def phase_c(v, acc):
            sl = pl.ds(v * L, L)
            w = [corn[c][sl] for c in range(8)]
            fx = fbuf[0][sl]
            fy = fbuf[1][sl]
            fz = fbuf[2][sl]
            gx = 1.0 - fx
            gy = 1.0 - fy
            gz = 1.0 - fz
            w00 = gy * gz
            w01 = gy * fz
            w10 = fy * gz
            w11 = fy * fz
            p0 = w[0] * w00 + w[1] * w01 + w[2] * w10 + w[3] * w11
            p1 = w[4] * w00 + w[5] * w01 + w[6] * w10 + w[7] * w11
            dss = gx * p0 + fx * p1
            n0v = p1 - p0
            q0 = (w[2] - w[0]) * gz + (w[3] - w[1]) * fz
            q1 = (w[6] - w[4]) * gz + (w[7] - w[5]) * fz
            n1v = gx * q0 + fx * q1
            r0 = (w[1] - w[0]) * gy + (w[3] - w[2]) * fy
            r1 = (w[5] - w[4]) * gy + (w[7] - w[6]) * fy
            n2v = gx * r0 + fx * r1
            ss = n0v * n0v + n1v * n1v + n2v * n2v
            inv = jnp.minimum(_rsqrt_newton(ss), 1e5)
            dbuf[sl] = dss
            n0b[sl] = n0v * inv
            n1b[sl] = n1v * inv
            n2b[sl] = n2v * inv
            return acc + jnp.minimum(dss, 0.0)

        acc = jnp.zeros((L,), jnp.float32)
        lax.fori_loop(0, VC, phase_a, 0, unroll=False)
        fire(0)
        for ch in range(1, NCH):
            lax.fori_loop(ch * VC, (ch + 1) * VC, phase_a, 0, unroll=False)
            fire(ch)
            drain(ch - 1)
            acc = lax.fori_loop((ch - 1) * VC, ch * VC, phase_c, acc,
                                unroll=False)
        drain(NCH - 1)
        acc = lax.fori_loop((NCH - 1) * VC, NCH * VC, phase_c, acc,
                            unroll=False)
        lbuf[...] = acc

        pltpu.sync_copy(dbuf, dss_hbm.at[pl.ds(b * N + n0, K)])
        for a, nb in enumerate((n0b, n1b, n2b)):
            pltpu.sync_copy(nb, nss_hbm.at[pl.ds((b * 3 + a) * N + n0, K)])
        pltpu.sync_copy(lbuf, lpart_hbm.at[pl.ds(wid * L, L)])

    return sc_kernel


def kernel(pss, sdf_grid, first, coef, max_limit):
    B, _, N = pss.shape
    G = sdf_grid.shape[-1]
    grid_flat = sdf_grid.reshape(B * G * G * G)
    pss_flat = pss.reshape(B * 3 * N)
    params = jnp.stack([first, coef, max_limit], axis=1)
    params = jnp.broadcast_to(params[..., None], (B, 3, 3, L))
    params = params.astype(jnp.float32).reshape(B * 9 * L)
    sc = _make_sc_kernel(B, N, G)
    dss_flat, nss_flat, lpart = sc(pss_flat, grid_flat, params)
    dss = dss_flat.reshape(B, N)
    nss = jnp.transpose(nss_flat.reshape(B, 3, N), (0, 2, 1))
    loss = -jnp.sum(lpart)
    return dss, nss, loss


# speedup vs baseline: 1.0200x; 1.0200x over previous
import functools

import jax
import jax.numpy as jnp
from jax import lax
from jax.experimental import pallas as pl
from jax.experimental.pallas import tpu as pltpu
from jax.experimental.pallas import tpu_sc as plsc

L = 16


def _rsqrt_newton(ss):
    i = lax.bitcast_convert_type(ss, jnp.int32)
    i = jnp.int32(0x5F3759DF) - lax.shift_right_logical(i, 1)
    y = lax.bitcast_convert_type(i, jnp.float32)
    half = ss * 0.5
    for _ in range(3):
        y = y * (1.5 - (half * y) * y)
    return y


def _make_sc_kernel(B, N, G):
    info = plsc.get_sparse_core_info()
    NC, NS = info.num_cores, info.num_subcores
    NW = NC * NS
    total = B * N
    K = total // NW
    tiles_per_b = N // K
    CS = 1024
    NCH = K // CS
    VC = CS // L
    G3 = G * G * G
    mesh = plsc.VectorSubcoreMesh(core_axis_name="c", subcore_axis_name="s")

    COFF = [(i * G * G + j * G + k) for i in (0, 1) for j in (0, 1) for k in (0, 1)]

    @functools.partial(
        pl.kernel,
        mesh=mesh,
        out_type=[
            jax.ShapeDtypeStruct((B * N,), jnp.float32),
            jax.ShapeDtypeStruct((B * 3 * N,), jnp.float32),
            jax.ShapeDtypeStruct((NW * L,), jnp.float32),
        ],
        scratch_types=(
            [pltpu.VMEM((K,), jnp.float32) for _ in range(3)]
            + [pltpu.VMEM((K,), jnp.float32) for _ in range(3)]
            + [pltpu.VMEM((K,), jnp.int32) for _ in range(8)]
            + [pltpu.VMEM((K,), jnp.float32) for _ in range(8)]
            + [pltpu.VMEM((K,), jnp.float32)]
            + [pltpu.VMEM((K,), jnp.float32) for _ in range(3)]
            + [pltpu.VMEM((9 * L,), jnp.float32)]
            + [pltpu.VMEM((L,), jnp.float32)]
            + [pltpu.SemaphoreType.DMA for _ in range(4)]
        ),
    )
    def sc_kernel(pss_hbm, grid_hbm, params_hbm,
                  dss_hbm, nss_hbm, lpart_hbm,
                  cx, cy, cz, fx_, fy_, fz_,
                  i0, i1, i2, i3, i4, i5, i6, i7,
                  g0, g1, g2, g3, g4, g5, g6, g7,
                  dbuf, n0b, n1b, n2b, pbuf, lbuf,
                  sem0, sem1, sem2, sem3):
        sems = (sem0, sem1, sem2, sem3)
        cbuf = (cx, cy, cz)
        fbuf = (fx_, fy_, fz_)
        idxb = (i0, i1, i2, i3, i4, i5, i6, i7)
        corn = (g0, g1, g2, g3, g4, g5, g6, g7)

        wid = lax.axis_index("s") * NC + lax.axis_index("c")
        b = wid // tiles_per_b
        n0 = (wid % tiles_per_b) * K

        for a in range(3):
            pltpu.sync_copy(pss_hbm.at[pl.ds((b * 3 + a) * N + n0, K)], cbuf[a])
        pltpu.sync_copy(params_hbm.at[pl.ds(b * 9 * L, 9 * L)], pbuf)

        gbase = b * G3

        def phase_a(v, _):
            sl = pl.ds(v * L, L)
            ib = []
            for a in range(3):
                p = cbuf[a][sl]
                f0 = jnp.maximum(
                    jnp.minimum((p - pbuf[pl.ds(a * L, L)]) * pbuf[pl.ds((3 + a) * L, L)],
                                pbuf[pl.ds((6 + a) * L, L)]), 0.0)
                ia = f0.astype(jnp.int32)
                fbuf[a][sl] = f0 - ia.astype(jnp.float32)
                ib.append(ia)
            flat = (ib[0] * G + ib[1]) * G + ib[2] + gbase
            for c in range(8):
                idxb[c][sl] = flat + COFF[c]
            return 0

        def fire(ch):
            cs = pl.ds(ch * CS, CS)
            for c in range(8):
                pltpu.async_copy(grid_hbm.at[idxb[c].at[cs]], corn[c].at[cs],
                                 sems[ch])

        def drain(ch):
            cs = pl.ds(ch * CS, CS)
            for c in range(8):
                pltpu.make_async_copy(
                    grid_hbm.at[idxb[c].at[cs]], corn[c].at[cs],
                    sems[ch]).wait()

        def phase_c(v, acc):
            sl = pl.ds(v * L, L)
            w = [corn[c][sl] for c in range(8)]
            fx = fbuf[0][sl]
            fy = fbuf[1][sl]
            fz = fbuf[2][sl]
            gx = 1.0 - fx
            gy = 1.0 - fy
            gz = 1.0 - fz
            w00 = gy * gz
            w01 = gy * fz
            w10 = fy * gz
            w11 = fy * fz
            p0 = w[0] * w00 + w[1] * w01 + w[2] * w10 + w[3] * w11
            p1 = w[4] * w00 + w[5] * w01 + w[6] * w10 + w[7] * w11
            dss = gx * p0 + fx * p1
            n0v = p1 - p0
            q0 = (w[2] - w[0]) * gz + (w[3] - w[1]) * fz
            q1 = (w[6] - w[4]) * gz + (w[7] - w[5]) * fz
            n1v = gx * q0 + fx * q1
            r0 = (w[1] - w[0]) * gy + (w[3] - w[2]) * fy
            r1 = (w[5] - w[4]) * gy + (w[7] - w[6]) * fy
            n2v = gx * r0 + fx * r1
            ss = n0v * n0v + n1v * n1v + n2v * n2v
            inv = jnp.minimum(_rsqrt_newton(ss), 1e5)
            dbuf[sl] = dss
            n0b[sl] = n0v * inv
            n1b[sl] = n1v * inv
            n2b[sl] = n2v * inv
            return acc + jnp.minimum(dss, 0.0)

        acc = jnp.zeros((L,), jnp.float32)
        lax.fori_loop(0, VC, phase_a, 0, unroll=False)
        fire(0)
        for ch in range(1, NCH):
            lax.fori_loop(ch * VC, (ch + 1) * VC, phase_a, 0, unroll=False)
            fire(ch)
            drain(ch - 1)
            acc = lax.fori_loop((ch - 1) * VC, ch * VC, phase_c, acc,
                                unroll=False)
        drain(NCH - 1)
        acc = lax.fori_loop((NCH - 1) * VC, NCH * VC, phase_c, acc,
                            unroll=False)
        lbuf[...] = acc

        pltpu.sync_copy(dbuf, dss_hbm.at[pl.ds(b * N + n0, K)])
        for a, nb in enumerate((n0b, n1b, n2b)):
            pltpu.sync_copy(nb, nss_hbm.at[pl.ds((b * 3 + a) * N + n0, K)])
        pltpu.sync_copy(lbuf, lpart_hbm.at[pl.ds(wid * L, L)])

    return sc_kernel


def kernel(pss, sdf_grid, first, coef, max_limit):
    B, _, N = pss.shape
    G = sdf_grid.shape[-1]
    grid_flat = sdf_grid.reshape(B * G * G * G)
    pss_flat = pss.reshape(B * 3 * N)
    params = jnp.stack([first, coef, max_limit], axis=1)
    params = jnp.broadcast_to(params[..., None], (B, 3, 3, L))
    params = params.astype(jnp.float32).reshape(B * 9 * L)
    sc = _make_sc_kernel(B, N, G)
    dss_flat, nss_flat, lpart = sc(pss_flat, grid_flat, params)
    dss = dss_flat.reshape(B, N)
    nss = jnp.transpose(nss_flat.reshape(B, 3, N), (0, 2, 1))
    loss = -jnp.sum(lpart)
    return dss, nss, loss
